# baseline (speedup 1.0000x reference)
import functools
import numpy as np
import jax
import jax.numpy as jnp
from jax import lax
from jax.experimental import pallas as pl
from jax.experimental.pallas import tpu as pltpu
from jax.experimental.pallas import tpu_sc as plsc

_ALPHA = 4
_NC, _NS = 2, 16


def _pack_sc(frames, T, n_slow, C, H, W, h_chunk):
    cpf = H // h_chunk
    n_chunks = C * T * cpf
    NW = _NC * _NS
    assert n_chunks % NW == 0
    per_worker = n_chunks // NW
    mesh = plsc.VectorSubcoreMesh(
        core_axis_name="c", subcore_axis_name="s", num_cores=_NC, num_subcores=_NS
    )

    @functools.partial(
        pl.kernel,
        out_type=[
            jax.ShapeDtypeStruct((C, n_slow, H, W), jnp.float32),
            jax.ShapeDtypeStruct((C, T, H, W), jnp.float32),
        ],
        mesh=mesh,
        scratch_types=[
            pltpu.VMEM((h_chunk, W), jnp.float32),
            pltpu.VMEM((h_chunk, W), jnp.float32),
            pltpu.SemaphoreType.DMA,
            pltpu.SemaphoreType.DMA,
        ],
    )
    def k(in_hbm, slow_hbm, fast_hbm, buf0, buf1, sem0, sem1):
        wid = lax.axis_index("s") * _NC + lax.axis_index("c")
        base = wid * per_worker
        bufs = (buf0, buf1)
        sems = (sem0, sem1)

        def coords(m):
            c = m // (T * cpf)
            rem = m % (T * cpf)
            t = rem // cpf
            sub = rem % cpf
            h0 = sub * h_chunk
            j = ((t + 1) * (n_slow - 1) - 1) // (T - 1)
            sel = (j * (T - 1)) // (n_slow - 1) == t
            return c, t, h0, j, sel

        def in_at(m):
            c, t, h0, j, sel = coords(m)
            return in_hbm.at[c, t, pl.ds(h0, h_chunk), :]

        pltpu.make_async_copy(in_at(base), bufs[0], sems[0]).start()
        for kk in range(per_worker):
            m = base + kk
            b = kk % 2
            if kk + 1 < per_worker:
                nb = (kk + 1) % 2
                pltpu.make_async_copy(in_at(m + 1), bufs[nb], sems[nb]).start()
            pltpu.make_async_copy(in_at(m), bufs[b], sems[b]).wait()
            c, t, h0, j, sel = coords(m)
            pltpu.sync_copy(bufs[b], fast_hbm.at[c, t, pl.ds(h0, h_chunk), :])

            @pl.when(sel)
            def _():
                pltpu.sync_copy(bufs[b], slow_hbm.at[c, j, pl.ds(h0, h_chunk), :])

    return k(frames)


def kernel(frames):
    C, T, H, W = frames.shape
    n_slow = T // _ALPHA
    idx = np.linspace(0.0, T - 1, n_slow).astype(np.int32)
    assert all(int(v) == (j * (T - 1)) // (n_slow - 1) for j, v in enumerate(idx))
    idx_set = set(idx.tolist())
    for t in range(T):
        j = ((t + 1) * (n_slow - 1) - 1) // (T - 1)
        assert ((j * (T - 1)) // (n_slow - 1) == t) == (t in idx_set)

    slow, fast = _pack_sc(frames, T, n_slow, C, H, W, h_chunk=96)
    return (slow, fast)


# speedup vs baseline: 1.4729x; 1.4729x over previous
import numpy as np
import jax
import jax.numpy as jnp
from jax.experimental import pallas as pl
from jax.experimental.pallas import tpu as pltpu

_ALPHA = 4
_BLK = 8


def _make_body(T, n_slow):
    def body(in_ref, fast_ref, slow_ref):
        b = pl.program_id(0)
        fast_ref[...] = in_ref[...]
        off0 = ((2 * b) * (T - 1)) // (n_slow - 1) - _BLK * b
        off1 = ((2 * b + 1) * (T - 1)) // (n_slow - 1) - _BLK * b
        slow_ref[:, pl.ds(0, 1)] = in_ref[:, pl.ds(off0, 1)]
        slow_ref[:, pl.ds(1, 1)] = in_ref[:, pl.ds(off1, 1)]

    return body


def kernel(frames):
    C, T, H, W = frames.shape
    n_slow = T // _ALPHA
    idx = np.linspace(0.0, T - 1, n_slow).astype(np.int32)
    assert all(int(v) == (j * (T - 1)) // (n_slow - 1) for j, v in enumerate(idx))
    for j, v in enumerate(idx):
        assert (j // 2) * _BLK <= int(v) < (j // 2 + 1) * _BLK

    fast, slow = pl.pallas_call(
        _make_body(T, n_slow),
        grid=(T // _BLK,),
        in_specs=[
            pl.BlockSpec((C, _BLK, H, W), lambda b: (0, b, 0, 0)),
        ],
        out_specs=[
            pl.BlockSpec((C, _BLK, H, W), lambda b: (0, b, 0, 0)),
            pl.BlockSpec((C, 2, H, W), lambda b: (0, b, 0, 0)),
        ],
        out_shape=[
            jax.ShapeDtypeStruct((C, T, H, W), frames.dtype),
            jax.ShapeDtypeStruct((C, n_slow, H, W), frames.dtype),
        ],
        compiler_params=pltpu.CompilerParams(vmem_limit_bytes=100 * 1024 * 1024),
    )(frames)

    return (slow, fast)
